# Initial kernel scaffold; baseline (speedup 1.0000x reference)
import functools
import jax
import jax.numpy as jnp
from jax import lax
from jax.experimental import pallas as pl
from jax.experimental.pallas import tpu as pltpu
from jax.experimental.pallas import tpu_sc as plsc

NC = 2
NS = 16
L = 16
NW = NC * NS
KSEG = 1024
STRIP = 16384


def _pool(B, C, HW):
  P = B * C
  PPW = P // NW
  mesh = plsc.VectorSubcoreMesh(core_axis_name="c", subcore_axis_name="s")

  @functools.partial(
      pl.kernel,
      mesh=mesh,
      out_type=jax.ShapeDtypeStruct((P, KSEG), jnp.float32),
      scratch_types=[
          pltpu.VMEM((STRIP,), jnp.int32),
          pltpu.VMEM((STRIP,), jnp.float32),
          pltpu.VMEM((L * KSEG,), jnp.float32),
          pltpu.VMEM((L * KSEG,), jnp.float32),
          pltpu.VMEM((KSEG,), jnp.float32),
      ],
  )
  def k(img_hbm, spx_hbm, out_hbm, lbl_v, data_v, acc0_v, acc1_v, row_v):
    wid = lax.axis_index("s") * NC + lax.axis_index("c")
    lane = lax.iota(jnp.int32, L)
    lane_k = lane * KSEG
    neg_inf = jnp.full((L,), -jnp.inf, jnp.float32)

    for i in range(PPW):
      plane = wid * PPW + i
      b = plane // C

      def init_body(j, _):
        acc0_v[pl.ds(j * L, L)] = neg_inf
        acc1_v[pl.ds(j * L, L)] = neg_inf
        return 0

      lax.fori_loop(0, KSEG, init_body, 0)

      def strip_body(s, _):
        pltpu.sync_copy(spx_hbm.at[b, pl.ds(s * STRIP, STRIP)], lbl_v)
        pltpu.sync_copy(img_hbm.at[plane, pl.ds(s * STRIP, STRIP)], data_v)

        def group_body(t, _):
          g0 = t * 2 * L
          lbl0 = lbl_v[pl.ds(g0, L)]
          v0 = data_v[pl.ds(g0, L)]
          idx0 = lane_k + lbl0
          cur0 = plsc.load_gather(acc0_v, [idx0])
          plsc.store_scatter(acc0_v, [idx0], jnp.maximum(cur0, v0))
          lbl1 = lbl_v[pl.ds(g0 + L, L)]
          v1 = data_v[pl.ds(g0 + L, L)]
          idx1 = lane_k + lbl1
          cur1 = plsc.load_gather(acc1_v, [idx1])
          plsc.store_scatter(acc1_v, [idx1], jnp.maximum(cur1, v1))
          return 0

        lax.fori_loop(0, STRIP // (2 * L), group_body, 0)
        return 0

      lax.fori_loop(0, HW // STRIP, strip_body, 0)

      def fin_body(j, _):
        m = jnp.maximum(acc0_v[pl.ds(j * L, L)], acc1_v[pl.ds(j * L, L)])
        for l in range(1, L):
          m = jnp.maximum(m, acc0_v[pl.ds(l * KSEG + j * L, L)])
          m = jnp.maximum(m, acc1_v[pl.ds(l * KSEG + j * L, L)])
        row_v[pl.ds(j * L, L)] = m
        return 0

      lax.fori_loop(0, KSEG // L, fin_body, 0)
      pltpu.sync_copy(row_v, out_hbm.at[plane])

  return k


@jax.jit
def kernel(img, spx):
  B, C, H, W = img.shape
  HW = H * W
  img2 = img.reshape(B * C, HW)
  spx2 = spx.reshape(B, HW).astype(jnp.int32)
  out = _pool(B, C, HW)(img2, spx2)
  return out.reshape(B, C, KSEG)


# speedup vs baseline: 1.3934x; 1.3934x over previous
import functools
import jax
import jax.numpy as jnp
from jax import lax
from jax.experimental import pallas as pl
from jax.experimental.pallas import tpu as pltpu
from jax.experimental.pallas import tpu_sc as plsc

NC = 2
NS = 16
L = 16
NW = NC * NS
KSEG = 1024
STRIP = 16384


def _pool(B, C, HW):
  P = B * C
  PPW = P // NW
  mesh = plsc.VectorSubcoreMesh(core_axis_name="c", subcore_axis_name="s")

  @functools.partial(
      pl.kernel,
      mesh=mesh,
      out_type=jax.ShapeDtypeStruct((P, KSEG), jnp.float32),
      compiler_params=pltpu.CompilerParams(
          needs_layout_passes=False, use_tc_tiling_on_sc=False
      ),
      scratch_types=[
          pltpu.VMEM((STRIP,), jnp.int32),
          pltpu.VMEM((STRIP,), jnp.float32),
          pltpu.VMEM((L * KSEG,), jnp.float32),
          pltpu.VMEM((L * KSEG,), jnp.float32),
          pltpu.VMEM((KSEG,), jnp.float32),
      ],
  )
  def k(img_hbm, spx_hbm, out_hbm, lbl_v, data_v, acc0_v, acc1_v, row_v):
    wid = lax.axis_index("s") * NC + lax.axis_index("c")
    lane = lax.iota(jnp.int32, L)
    lane_k = lane * KSEG
    neg_inf = jnp.full((L,), -jnp.inf, jnp.float32)

    for i in range(PPW):
      plane = wid * PPW + i
      b = plane // C

      def init_body(j, _):
        acc0_v[pl.ds(j * L, L)] = neg_inf
        acc1_v[pl.ds(j * L, L)] = neg_inf
        return 0

      lax.fori_loop(0, KSEG, init_body, 0)

      def strip_body(s, _):
        pltpu.sync_copy(spx_hbm.at[b, pl.ds(s * STRIP, STRIP)], lbl_v)
        pltpu.sync_copy(img_hbm.at[plane, pl.ds(s * STRIP, STRIP)], data_v)

        def group_body(t, _):
          g0 = t * 2 * L
          lbl0 = lbl_v[pl.ds(g0, L)]
          v0 = data_v[pl.ds(g0, L)]
          idx0 = lane_k + lbl0
          cur0 = plsc.load_gather(acc0_v, [idx0])
          plsc.store_scatter(acc0_v, [idx0], jnp.maximum(cur0, v0))
          lbl1 = lbl_v[pl.ds(g0 + L, L)]
          v1 = data_v[pl.ds(g0 + L, L)]
          idx1 = lane_k + lbl1
          cur1 = plsc.load_gather(acc1_v, [idx1])
          plsc.store_scatter(acc1_v, [idx1], jnp.maximum(cur1, v1))
          return 0

        lax.fori_loop(0, STRIP // (2 * L), group_body, 0)
        return 0

      lax.fori_loop(0, HW // STRIP, strip_body, 0)

      def fin_body(j, _):
        m = jnp.maximum(acc0_v[pl.ds(j * L, L)], acc1_v[pl.ds(j * L, L)])
        for l in range(1, L):
          m = jnp.maximum(m, acc0_v[pl.ds(l * KSEG + j * L, L)])
          m = jnp.maximum(m, acc1_v[pl.ds(l * KSEG + j * L, L)])
        row_v[pl.ds(j * L, L)] = m
        return 0

      lax.fori_loop(0, KSEG // L, fin_body, 0)
      pltpu.sync_copy(row_v, out_hbm.at[plane])

  return k


@jax.jit
def kernel(img, spx):
  B, C, H, W = img.shape
  HW = H * W
  img2 = img.reshape(B * C, HW)
  spx2 = spx.reshape(B, HW).astype(jnp.int32)
  out = _pool(B, C, HW)(img2, spx2)
  return out.reshape(B, C, KSEG)


# speedup vs baseline: 4.0518x; 2.9079x over previous
import functools
import jax
import jax.numpy as jnp
from jax import lax
from jax.experimental import pallas as pl
from jax.experimental.pallas import tpu as pltpu
from jax.experimental.pallas import tpu_sc as plsc

NC = 2
NS = 16
L = 16
NW = NC * NS
KSEG = 1024
STRIP = 8192
UNROLL = 4


def _pool(B, C, HW):
  P = B * C
  PPW = P // NW
  NPASS = PPW // 2
  NSTRIP = HW // STRIP
  mesh = plsc.VectorSubcoreMesh(core_axis_name="c", subcore_axis_name="s")

  @functools.partial(
      pl.kernel,
      mesh=mesh,
      out_type=jax.ShapeDtypeStruct((P, KSEG), jnp.float32),
      compiler_params=pltpu.CompilerParams(
          needs_layout_passes=False, use_tc_tiling_on_sc=False
      ),
      scratch_types=[
          pltpu.VMEM((2, STRIP), jnp.int32),
          pltpu.VMEM((2, STRIP), jnp.float32),
          pltpu.VMEM((2, STRIP), jnp.float32),
          pltpu.VMEM((L * KSEG,), jnp.float32),
          pltpu.VMEM((L * KSEG,), jnp.float32),
          pltpu.VMEM((L * KSEG,), jnp.float32),
          pltpu.VMEM((L * KSEG,), jnp.float32),
          pltpu.VMEM((KSEG,), jnp.float32),
          pltpu.SemaphoreType.DMA,
          pltpu.SemaphoreType.DMA,
      ],
  )
  def k(img_hbm, spx_hbm, out_hbm, lbl_v, d0_v, d1_v,
        a00_v, a01_v, a10_v, a11_v, row_v, sem0, sem1):
    wid = lax.axis_index("s") * NC + lax.axis_index("c")
    lane = lax.iota(jnp.int32, L)
    lane_k = lane * KSEG
    neg_inf = jnp.full((L,), -jnp.inf, jnp.float32)
    sems = (sem0, sem1)

    def issue(s, slot, p0, p1, b):
      off = s * STRIP
      pltpu.async_copy(
          spx_hbm.at[b, pl.ds(off, STRIP)], lbl_v.at[slot], sems[slot])
      pltpu.async_copy(
          img_hbm.at[p0, pl.ds(off, STRIP)], d0_v.at[slot], sems[slot])
      pltpu.async_copy(
          img_hbm.at[p1, pl.ds(off, STRIP)], d1_v.at[slot], sems[slot])

    def wait(slot):
      pltpu.make_async_copy(
          spx_hbm.at[0, pl.ds(0, STRIP)], lbl_v.at[slot], sems[slot]).wait()
      pltpu.make_async_copy(
          img_hbm.at[0, pl.ds(0, STRIP)], d0_v.at[slot], sems[slot]).wait()
      pltpu.make_async_copy(
          img_hbm.at[0, pl.ds(0, STRIP)], d1_v.at[slot], sems[slot]).wait()

    for ps in range(NPASS):
      p0 = wid * PPW + 2 * ps
      p1 = p0 + 1
      b = p0 // C

      def init_body(j, _):
        o = j * (4 * L)
        for u in range(4):
          a00_v[pl.ds(o + u * L, L)] = neg_inf
          a01_v[pl.ds(o + u * L, L)] = neg_inf
          a10_v[pl.ds(o + u * L, L)] = neg_inf
          a11_v[pl.ds(o + u * L, L)] = neg_inf
        return 0

      lax.fori_loop(0, KSEG // 4, init_body, 0)

      issue(0, 0, p0, p1, b)

      def process(slot):
        def group_body(t, _):
          base = t * (2 * UNROLL * L)
          for u in range(UNROLL):
            o_e = base + 2 * u * L
            o_o = o_e + L
            lbl_e = lbl_v[slot, pl.ds(o_e, L)]
            lbl_o = lbl_v[slot, pl.ds(o_o, L)]
            idx_e = lane_k + lbl_e
            idx_o = lane_k + lbl_o
            v0e = d0_v[slot, pl.ds(o_e, L)]
            v0o = d0_v[slot, pl.ds(o_o, L)]
            v1e = d1_v[slot, pl.ds(o_e, L)]
            v1o = d1_v[slot, pl.ds(o_o, L)]
            c00 = plsc.load_gather(a00_v, [idx_e])
            c01 = plsc.load_gather(a01_v, [idx_o])
            c10 = plsc.load_gather(a10_v, [idx_e])
            c11 = plsc.load_gather(a11_v, [idx_o])
            plsc.store_scatter(a00_v, [idx_e], jnp.maximum(c00, v0e))
            plsc.store_scatter(a01_v, [idx_o], jnp.maximum(c01, v0o))
            plsc.store_scatter(a10_v, [idx_e], jnp.maximum(c10, v1e))
            plsc.store_scatter(a11_v, [idx_o], jnp.maximum(c11, v1o))
          return 0

        lax.fori_loop(0, STRIP // (2 * UNROLL * L), group_body, 0)

      def strip_body(s2, _):
        s = s2 * 2
        issue(s + 1, 1, p0, p1, b)
        wait(0)
        process(0)

        @pl.when(s2 + 1 < NSTRIP // 2)
        def _():
          issue(s + 2, 0, p0, p1, b)

        wait(1)
        process(1)
        return 0

      lax.fori_loop(0, NSTRIP // 2, strip_body, 0)

      def fin0_body(j, _):
        m = jnp.maximum(a00_v[pl.ds(j * L, L)], a01_v[pl.ds(j * L, L)])
        for l in range(1, L):
          m = jnp.maximum(m, a00_v[pl.ds(l * KSEG + j * L, L)])
          m = jnp.maximum(m, a01_v[pl.ds(l * KSEG + j * L, L)])
        row_v[pl.ds(j * L, L)] = m
        return 0

      lax.fori_loop(0, KSEG // L, fin0_body, 0)
      pltpu.sync_copy(row_v, out_hbm.at[p0])

      def fin1_body(j, _):
        m = jnp.maximum(a10_v[pl.ds(j * L, L)], a11_v[pl.ds(j * L, L)])
        for l in range(1, L):
          m = jnp.maximum(m, a10_v[pl.ds(l * KSEG + j * L, L)])
          m = jnp.maximum(m, a11_v[pl.ds(l * KSEG + j * L, L)])
        row_v[pl.ds(j * L, L)] = m
        return 0

      lax.fori_loop(0, KSEG // L, fin1_body, 0)
      pltpu.sync_copy(row_v, out_hbm.at[p1])

  return k


@jax.jit
def kernel(img, spx):
  B, C, H, W = img.shape
  HW = H * W
  img2 = img.reshape(B * C, HW)
  spx2 = spx.reshape(B, HW).astype(jnp.int32)
  out = _pool(B, C, HW)(img2, spx2)
  return out.reshape(B, C, KSEG)


# speedup vs baseline: 4.0536x; 1.0004x over previous
import functools
import jax
import jax.numpy as jnp
from jax import lax
from jax.experimental import pallas as pl
from jax.experimental.pallas import tpu as pltpu
from jax.experimental.pallas import tpu_sc as plsc

NC = 2
NS = 16
L = 16
NW = NC * NS
KSEG = 1024
STRIP = 8192
UNROLL = 8


def _pool(B, C, HW):
  P = B * C
  PPW = P // NW
  NPASS = PPW // 2
  NSTRIP = HW // STRIP
  mesh = plsc.VectorSubcoreMesh(core_axis_name="c", subcore_axis_name="s")

  @functools.partial(
      pl.kernel,
      mesh=mesh,
      out_type=jax.ShapeDtypeStruct((P, KSEG), jnp.float32),
      compiler_params=pltpu.CompilerParams(
          needs_layout_passes=False, use_tc_tiling_on_sc=False
      ),
      scratch_types=[
          pltpu.VMEM((2, STRIP), jnp.int32),
          pltpu.VMEM((2, STRIP), jnp.float32),
          pltpu.VMEM((2, STRIP), jnp.float32),
          pltpu.VMEM((L * KSEG,), jnp.float32),
          pltpu.VMEM((L * KSEG,), jnp.float32),
          pltpu.VMEM((L * KSEG,), jnp.float32),
          pltpu.VMEM((L * KSEG,), jnp.float32),
          pltpu.VMEM((KSEG,), jnp.float32),
          pltpu.SemaphoreType.DMA,
          pltpu.SemaphoreType.DMA,
      ],
  )
  def k(img_hbm, spx_hbm, out_hbm, lbl_v, d0_v, d1_v,
        a00_v, a01_v, a10_v, a11_v, row_v, sem0, sem1):
    wid = lax.axis_index("s") * NC + lax.axis_index("c")
    lane = lax.iota(jnp.int32, L)
    lane_k = lane * KSEG
    neg_inf = jnp.full((L,), -jnp.inf, jnp.float32)
    sems = (sem0, sem1)

    def issue(s, slot, p0, p1, b):
      off = s * STRIP
      pltpu.async_copy(
          spx_hbm.at[b, pl.ds(off, STRIP)], lbl_v.at[slot], sems[slot])
      pltpu.async_copy(
          img_hbm.at[p0, pl.ds(off, STRIP)], d0_v.at[slot], sems[slot])
      pltpu.async_copy(
          img_hbm.at[p1, pl.ds(off, STRIP)], d1_v.at[slot], sems[slot])

    def wait(slot):
      pltpu.make_async_copy(
          spx_hbm.at[0, pl.ds(0, STRIP)], lbl_v.at[slot], sems[slot]).wait()
      pltpu.make_async_copy(
          img_hbm.at[0, pl.ds(0, STRIP)], d0_v.at[slot], sems[slot]).wait()
      pltpu.make_async_copy(
          img_hbm.at[0, pl.ds(0, STRIP)], d1_v.at[slot], sems[slot]).wait()

    for ps in range(NPASS):
      p0 = wid * PPW + 2 * ps
      p1 = p0 + 1
      b = p0 // C

      def init_body(j, _):
        o = j * (4 * L)
        for u in range(4):
          a00_v[pl.ds(o + u * L, L)] = neg_inf
          a01_v[pl.ds(o + u * L, L)] = neg_inf
          a10_v[pl.ds(o + u * L, L)] = neg_inf
          a11_v[pl.ds(o + u * L, L)] = neg_inf
        return 0

      lax.fori_loop(0, KSEG // 4, init_body, 0)

      issue(0, 0, p0, p1, b)

      def process(slot):
        def group_body(t, _):
          base = t * (2 * UNROLL * L)
          for u in range(UNROLL):
            o_e = base + 2 * u * L
            o_o = o_e + L
            lbl_e = lbl_v[slot, pl.ds(o_e, L)]
            lbl_o = lbl_v[slot, pl.ds(o_o, L)]
            idx_e = lane_k + lbl_e
            idx_o = lane_k + lbl_o
            v0e = d0_v[slot, pl.ds(o_e, L)]
            v0o = d0_v[slot, pl.ds(o_o, L)]
            v1e = d1_v[slot, pl.ds(o_e, L)]
            v1o = d1_v[slot, pl.ds(o_o, L)]
            c00 = plsc.load_gather(a00_v, [idx_e])
            c01 = plsc.load_gather(a01_v, [idx_o])
            c10 = plsc.load_gather(a10_v, [idx_e])
            c11 = plsc.load_gather(a11_v, [idx_o])
            plsc.store_scatter(a00_v, [idx_e], jnp.maximum(c00, v0e))
            plsc.store_scatter(a01_v, [idx_o], jnp.maximum(c01, v0o))
            plsc.store_scatter(a10_v, [idx_e], jnp.maximum(c10, v1e))
            plsc.store_scatter(a11_v, [idx_o], jnp.maximum(c11, v1o))
          return 0

        lax.fori_loop(0, STRIP // (2 * UNROLL * L), group_body, 0)

      def strip_body(s2, _):
        s = s2 * 2
        issue(s + 1, 1, p0, p1, b)
        wait(0)
        process(0)

        @pl.when(s2 + 1 < NSTRIP // 2)
        def _():
          issue(s + 2, 0, p0, p1, b)

        wait(1)
        process(1)
        return 0

      lax.fori_loop(0, NSTRIP // 2, strip_body, 0)

      def fin0_body(j, _):
        m = jnp.maximum(a00_v[pl.ds(j * L, L)], a01_v[pl.ds(j * L, L)])
        for l in range(1, L):
          m = jnp.maximum(m, a00_v[pl.ds(l * KSEG + j * L, L)])
          m = jnp.maximum(m, a01_v[pl.ds(l * KSEG + j * L, L)])
        row_v[pl.ds(j * L, L)] = m
        return 0

      lax.fori_loop(0, KSEG // L, fin0_body, 0)
      pltpu.sync_copy(row_v, out_hbm.at[p0])

      def fin1_body(j, _):
        m = jnp.maximum(a10_v[pl.ds(j * L, L)], a11_v[pl.ds(j * L, L)])
        for l in range(1, L):
          m = jnp.maximum(m, a10_v[pl.ds(l * KSEG + j * L, L)])
          m = jnp.maximum(m, a11_v[pl.ds(l * KSEG + j * L, L)])
        row_v[pl.ds(j * L, L)] = m
        return 0

      lax.fori_loop(0, KSEG // L, fin1_body, 0)
      pltpu.sync_copy(row_v, out_hbm.at[p1])

  return k


@jax.jit
def kernel(img, spx):
  B, C, H, W = img.shape
  HW = H * W
  img2 = img.reshape(B * C, HW)
  spx2 = spx.reshape(B, HW).astype(jnp.int32)
  out = _pool(B, C, HW)(img2, spx2)
  return out.reshape(B, C, KSEG)


# speedup vs baseline: 4.0557x; 1.0005x over previous
import functools
import jax
import jax.numpy as jnp
from jax import lax
from jax.experimental import pallas as pl
from jax.experimental.pallas import tpu as pltpu
from jax.experimental.pallas import tpu_sc as plsc

NC = 2
NS = 16
L = 16
NW = NC * NS
KSEG = 1024
STRIP = 8192
UNROLL = 8


def _pool(B, C, HW):
  P = B * C
  PPW = P // NW
  NPASS = PPW // 2
  NSTRIP = HW // STRIP
  mesh = plsc.VectorSubcoreMesh(core_axis_name="c", subcore_axis_name="s")

  @functools.partial(
      pl.kernel,
      mesh=mesh,
      out_type=jax.ShapeDtypeStruct((P, KSEG), jnp.float32),
      compiler_params=pltpu.CompilerParams(
          needs_layout_passes=False, use_tc_tiling_on_sc=False
      ),
      scratch_types=[
          pltpu.VMEM((2, STRIP), jnp.int32),
          pltpu.VMEM((2, 2, STRIP), jnp.float32),
          pltpu.VMEM((L * KSEG,), jnp.float32),
          pltpu.VMEM((L * KSEG,), jnp.float32),
          pltpu.VMEM((L * KSEG,), jnp.float32),
          pltpu.VMEM((L * KSEG,), jnp.float32),
          pltpu.VMEM((KSEG,), jnp.float32),
          pltpu.SemaphoreType.DMA,
          pltpu.SemaphoreType.DMA,
      ],
  )
  def k(img_hbm, spx_hbm, out_hbm, lbl_v, d_v,
        a00_v, a01_v, a10_v, a11_v, row_v, sem0, sem1):
    wid = lax.axis_index("s") * NC + lax.axis_index("c")
    lane = lax.iota(jnp.int32, L)
    lane_k = lane * KSEG
    neg_inf = jnp.full((L,), -jnp.inf, jnp.float32)
    sems = (sem0, sem1)

    def issue(s, slot, p0, b):
      off = s * STRIP
      pltpu.async_copy(
          spx_hbm.at[b, pl.ds(off, STRIP)], lbl_v.at[slot], sems[slot])
      pltpu.async_copy(
          img_hbm.at[pl.ds(p0, 2), pl.ds(off, STRIP)], d_v.at[slot],
          sems[slot])

    def wait(slot):
      pltpu.make_async_copy(
          spx_hbm.at[0, pl.ds(0, STRIP)], lbl_v.at[slot], sems[slot]).wait()
      pltpu.make_async_copy(
          img_hbm.at[pl.ds(0, 2), pl.ds(0, STRIP)], d_v.at[slot],
          sems[slot]).wait()

    for ps in range(NPASS):
      p0 = wid * PPW + 2 * ps
      p1 = p0 + 1
      b = p0 // C

      def init_body(j, _):
        o = j * (4 * L)
        for u in range(4):
          a00_v[pl.ds(o + u * L, L)] = neg_inf
          a01_v[pl.ds(o + u * L, L)] = neg_inf
          a10_v[pl.ds(o + u * L, L)] = neg_inf
          a11_v[pl.ds(o + u * L, L)] = neg_inf
        return 0

      lax.fori_loop(0, KSEG // 4, init_body, 0)

      issue(0, 0, p0, b)

      def process(slot):
        def group_body(t, _):
          base = t * (2 * UNROLL * L)
          for u in range(UNROLL):
            o_e = base + 2 * u * L
            o_o = o_e + L
            lbl_e = lbl_v[slot, pl.ds(o_e, L)]
            lbl_o = lbl_v[slot, pl.ds(o_o, L)]
            idx_e = lane_k + lbl_e
            idx_o = lane_k + lbl_o
            v0e = d_v[slot, 0, pl.ds(o_e, L)]
            v0o = d_v[slot, 0, pl.ds(o_o, L)]
            v1e = d_v[slot, 1, pl.ds(o_e, L)]
            v1o = d_v[slot, 1, pl.ds(o_o, L)]
            c00 = plsc.load_gather(a00_v, [idx_e])
            c01 = plsc.load_gather(a01_v, [idx_o])
            c10 = plsc.load_gather(a10_v, [idx_e])
            c11 = plsc.load_gather(a11_v, [idx_o])
            plsc.store_scatter(a00_v, [idx_e], jnp.maximum(c00, v0e))
            plsc.store_scatter(a01_v, [idx_o], jnp.maximum(c01, v0o))
            plsc.store_scatter(a10_v, [idx_e], jnp.maximum(c10, v1e))
            plsc.store_scatter(a11_v, [idx_o], jnp.maximum(c11, v1o))
          return 0

        lax.fori_loop(0, STRIP // (2 * UNROLL * L), group_body, 0)

      def strip_body(s2, _):
        s = s2 * 2
        issue(s + 1, 1, p0, b)
        wait(0)
        process(0)

        @pl.when(s2 + 1 < NSTRIP // 2)
        def _():
          issue(s + 2, 0, p0, b)

        wait(1)
        process(1)
        return 0

      lax.fori_loop(0, NSTRIP // 2, strip_body, 0)

      def fin0_body(j, _):
        m = jnp.maximum(a00_v[pl.ds(j * L, L)], a01_v[pl.ds(j * L, L)])
        for l in range(1, L):
          m = jnp.maximum(m, a00_v[pl.ds(l * KSEG + j * L, L)])
          m = jnp.maximum(m, a01_v[pl.ds(l * KSEG + j * L, L)])
        row_v[pl.ds(j * L, L)] = m
        return 0

      lax.fori_loop(0, KSEG // L, fin0_body, 0)
      pltpu.sync_copy(row_v, out_hbm.at[p0])

      def fin1_body(j, _):
        m = jnp.maximum(a10_v[pl.ds(j * L, L)], a11_v[pl.ds(j * L, L)])
        for l in range(1, L):
          m = jnp.maximum(m, a10_v[pl.ds(l * KSEG + j * L, L)])
          m = jnp.maximum(m, a11_v[pl.ds(l * KSEG + j * L, L)])
        row_v[pl.ds(j * L, L)] = m
        return 0

      lax.fori_loop(0, KSEG // L, fin1_body, 0)
      pltpu.sync_copy(row_v, out_hbm.at[p1])

  return k


@jax.jit
def kernel(img, spx):
  B, C, H, W = img.shape
  HW = H * W
  img2 = img.reshape(B * C, HW)
  spx2 = spx.reshape(B, HW).astype(jnp.int32)
  out = _pool(B, C, HW)(img2, spx2)
  return out.reshape(B, C, KSEG)
